# baseline (speedup 1.0000x reference)
import functools

import jax
import jax.numpy as jnp
from jax import lax
from jax.experimental import pallas as pl
from jax.experimental.pallas import tpu as pltpu
from jax.experimental.pallas import tpu_sc as plsc

NC = 2
NS = 16
NW = NC * NS
CHUNK = 128
BT = 640
LANES = 128


def _gather_body(nrow, tok_hbm, ix_hbm, out_hbm, idx_v, buf0, buf1, s0, s1):
    wid = lax.axis_index("s") * NC + lax.axis_index("c")
    base = wid * nrow
    pltpu.sync_copy(ix_hbm.at[pl.ds(base, nrow)], idx_v)
    bufs = (buf0, buf1)
    sems = (s0, s1)
    copies = [None, None]
    copies[0] = pltpu.async_copy(tok_hbm.at[idx_v.at[0]], buf0, sems[0])
    for j in range(nrow):
        if j + 1 < nrow:
            copies[(j + 1) % 2] = pltpu.async_copy(
                tok_hbm.at[idx_v.at[j + 1]], bufs[(j + 1) % 2], sems[(j + 1) % 2]
            )
        copies[j % 2].wait()
        pltpu.sync_copy(bufs[j % 2], out_hbm.at[pl.ds((base + j) * CHUNK, CHUNK)])


def _sc_gather(token_emb128, ix_flat):
    n = ix_flat.shape[0]
    nrow = n // (NW * CHUNK)
    ix2 = ix_flat.reshape(NW * nrow, CHUNK)
    mesh = plsc.VectorSubcoreMesh(core_axis_name="c", subcore_axis_name="s")
    f = pl.kernel(
        functools.partial(_gather_body, nrow),
        out_type=jax.ShapeDtypeStruct((n, LANES), jnp.float32),
        mesh=mesh,
        scratch_types=[
            pltpu.VMEM((nrow, CHUNK), jnp.int32),
            pltpu.VMEM((CHUNK, LANES), jnp.float32),
            pltpu.VMEM((CHUNK, LANES), jnp.float32),
            pltpu.SemaphoreType.DMA,
            pltpu.SemaphoreType.DMA,
        ],
        compiler_params=pltpu.CompilerParams(use_tc_tiling_on_sc=False),
    )
    return f(token_emb128, ix2)


def _head_body(emb, x_ref, p_ref, w_ref, b_ref, o_ref):
    x = x_ref[:, :emb] + p_ref[...]
    o_ref[...] = (
        jnp.dot(x, w_ref[...], preferred_element_type=jnp.float32) + b_ref[...]
    )


def _tc_head(tok128, ptile, W, b2d):
    emb, vocab = W.shape
    n = tok128.shape[0]
    return pl.pallas_call(
        functools.partial(_head_body, emb),
        grid=(n // BT,),
        in_specs=[
            pl.BlockSpec((BT, LANES), lambda i: (i, 0)),
            pl.BlockSpec((BT, emb), lambda i: (0, 0)),
            pl.BlockSpec((emb, vocab), lambda i: (0, 0)),
            pl.BlockSpec((1, vocab), lambda i: (0, 0)),
        ],
        out_specs=pl.BlockSpec((BT, vocab), lambda i: (i, 0)),
        out_shape=jax.ShapeDtypeStruct((n, vocab), jnp.float32),
    )(tok128, ptile, W, b2d)


def kernel(ix, token_emb, pos_emb, W, b):
    bsz, tlen = ix.shape
    n = bsz * tlen
    emb = token_emb.shape[1]
    vocab = W.shape[1]
    te128 = jnp.pad(token_emb, ((0, 0), (0, LANES - emb)))
    tok128 = _sc_gather(te128, ix.reshape(n).astype(jnp.int32))
    ptile = jnp.tile(pos_emb, (BT // tlen, 1))
    logits2d = _tc_head(tok128, ptile, W, b.reshape(1, vocab))
    return logits2d.reshape(bsz, tlen, vocab)


# speedup vs baseline: 4.5276x; 4.5276x over previous
import functools

import jax
import jax.numpy as jnp
from jax import lax
from jax.experimental import pallas as pl
from jax.experimental.pallas import tpu as pltpu
from jax.experimental.pallas import tpu_sc as plsc

NC = 2
NS = 16
NW = NC * NS
TPAD = 24
CHUNK = 128
BTB = 4096
LANES = 128


NBUF = 4


def _gather_body(nchunk, tok_hbm, ix_hbm, out_hbm, idx_v, *rest):
    bufs, gsems, wsems = rest[:NBUF], rest[NBUF:2 * NBUF], rest[2 * NBUF:3 * NBUF]
    wid = lax.axis_index("s") * NC + lax.axis_index("c")
    base = wid * nchunk
    pltpu.sync_copy(ix_hbm.at[pl.ds(base, nchunk)], idx_v)
    gathers = [None] * NBUF
    writes = [None] * NBUF
    for j in range(min(NBUF, nchunk)):
        gathers[j] = pltpu.async_copy(tok_hbm.at[idx_v.at[j]], bufs[j], gsems[j])
    for j in range(nchunk):
        k = j % NBUF
        gathers[k].wait()
        writes[k] = pltpu.async_copy(
            bufs[k], out_hbm.at[pl.ds((base + j) * CHUNK, CHUNK)], wsems[k]
        )
        nxt = j + NBUF
        if nxt < nchunk:
            writes[k].wait()
            gathers[k] = pltpu.async_copy(
                tok_hbm.at[idx_v.at[nxt]], bufs[k], gsems[k]
            )
    for j in range(max(0, nchunk - NBUF), nchunk):
        writes[j % NBUF].wait()


def _sc_gather(token_emb128, ix2d, out_rows):
    nrow = ix2d.shape[0]
    nchunk = nrow // NW
    mesh = plsc.VectorSubcoreMesh(core_axis_name="c", subcore_axis_name="s")
    f = pl.kernel(
        functools.partial(_gather_body, nchunk),
        out_type=jax.ShapeDtypeStruct((out_rows, LANES), jnp.float32),
        mesh=mesh,
        scratch_types=(
            [pltpu.VMEM((nchunk, CHUNK), jnp.int32)]
            + [pltpu.VMEM((CHUNK, LANES), jnp.float32)] * NBUF
            + [pltpu.SemaphoreType.DMA] * (2 * NBUF)
        ),
        compiler_params=pltpu.CompilerParams(use_tc_tiling_on_sc=False),
    )
    return f(token_emb128, ix2d)


def _head_body(emb, x_ref, p_ref, w_ref, bt_ref, o_ref):
    x = x_ref[0, :, :emb]
    dot_t = lax.dot_general(
        w_ref[...], x, (((0,), (1,)), ((), ())),
        preferred_element_type=jnp.float32,
    )
    pos_col = lax.dot_general(
        w_ref[...], p_ref[0], (((0,), (1,)), ((), ())),
        preferred_element_type=jnp.float32,
    )
    o_ref[0] = dot_t + pos_col + bt_ref[...]


def _tc_head(tok3, pos3, W, bt):
    emb, vocab = W.shape
    bsz = tok3.shape[1]
    tlen = pos3.shape[0]
    return pl.pallas_call(
        functools.partial(_head_body, emb),
        grid=(bsz // BTB, tlen),
        in_specs=[
            pl.BlockSpec((1, BTB, LANES), lambda i, t: (t, i, 0)),
            pl.BlockSpec((1, 1, emb), lambda i, t: (t, 0, 0)),
            pl.BlockSpec((emb, vocab), lambda i, t: (0, 0)),
            pl.BlockSpec((vocab, 1), lambda i, t: (0, 0)),
        ],
        out_specs=pl.BlockSpec((1, vocab, BTB), lambda i, t: (t, 0, i)),
        out_shape=jax.ShapeDtypeStruct((tlen, vocab, bsz), jnp.float32),
    )(tok3, pos3, W, bt)


def kernel(ix, token_emb, pos_emb, W, b):
    bsz, tlen = ix.shape
    emb = token_emb.shape[1]
    vocab = W.shape[1]
    te128 = jnp.pad(token_emb, ((0, 0), (0, LANES - emb)))
    ix_t = ix.astype(jnp.int32).T
    tok2d = _sc_gather(
        te128, ix_t.reshape(bsz * tlen // CHUNK, CHUNK), bsz * tlen
    )
    tok3 = tok2d.reshape(tlen, bsz, LANES)
    logits_t = _tc_head(tok3, pos_emb.reshape(tlen, 1, emb), W, b.reshape(vocab, 1))
    return jnp.transpose(logits_t, (2, 0, 1))
